# baseline (speedup 1.0000x reference)
import functools

import jax
import jax.numpy as jnp
from jax import lax
from jax.experimental import pallas as pl
from jax.experimental.pallas import tpu as pltpu
from jax.experimental.pallas import tpu_sc as plsc

_RES = 32
_EPS = 1e-08
_B = 8
_N = 32768
_C = 64
_V = _RES ** 3

_K = 512
_KV = 2048
_GCH = 128

_SC_MESH = plsc.VectorSubcoreMesh(core_axis_name="c", subcore_axis_name="s")


@functools.partial(
    pl.kernel,
    mesh=_SC_MESH,
    compiler_params=pltpu.CompilerParams(needs_layout_passes=False),
    out_type=jax.ShapeDtypeStruct((_B * _V,), jnp.int32),
    scratch_types=[
        pltpu.VMEM((_N,), jnp.int32),
        pltpu.VMEM((_V,), jnp.int32),
    ],
)
def _sc_hist_cum(vox_hbm, cum_hbm, buf_v, hist_v):
    cid = lax.axis_index("c")
    sid = lax.axis_index("s")
    wid = sid * 2 + cid

    @pl.when(wid < _B)
    def _():
        b = wid
        pltpu.sync_copy(vox_hbm.at[pl.ds(b * _N, _N)], buf_v)

        zeros16 = jnp.zeros((16,), jnp.int32)

        def _zero(i, carry):
            hist_v[pl.ds(i * 16, 16)] = zeros16
            return carry

        lax.fori_loop(0, _V // 16, _zero, 0)

        ones16 = jnp.ones((16,), jnp.int32)
        lane = lax.broadcasted_iota(jnp.int32, (16,), 0)

        def _hist(i, carry):
            idx = buf_v[pl.ds(i * 16, 16)]
            for j in range(16):
                plsc.addupdate_scatter(hist_v, [idx], ones16, mask=lane == j)
            return carry

        lax.fori_loop(0, _N // 16, _hist, 0)

        def _cum(i, carry):
            v = hist_v[pl.ds(i * 16, 16)]
            cs = jnp.cumsum(v) + carry
            buf_v[pl.ds(i * 16, 16)] = cs
            return carry + jnp.sum(v)

        lax.fori_loop(0, _V // 16, _cum, jnp.int32(0))
        pltpu.sync_copy(buf_v, cum_hbm.at[pl.ds(b * _V, _V)])


@functools.partial(
    pl.kernel,
    mesh=_SC_MESH,
    compiler_params=pltpu.CompilerParams(
        needs_layout_passes=False, use_tc_tiling_on_sc=False
    ),
    out_type=jax.ShapeDtypeStruct((_B * _V, _C), jnp.float32),
    scratch_types=[
        pltpu.VMEM((_V // 4,), jnp.int32),
        pltpu.VMEM((4, _GCH), jnp.int32),
        pltpu.VMEM((4, _GCH, _C), jnp.float32),
        pltpu.SemaphoreType.DMA,
        pltpu.SemaphoreType.DMA,
        pltpu.SemaphoreType.DMA,
        pltpu.SemaphoreType.DMA,
        pltpu.SemaphoreType.DMA,
        pltpu.SemaphoreType.DMA,
        pltpu.SemaphoreType.DMA,
        pltpu.SemaphoreType.DMA,
    ],
)
def _sc_gather(s_hbm, cum_hbm, g_hbm, cum_v, idx_v, rows_v,
               g0, g1, g2, g3, w0, w1, w2, w3):
    gsem = (g0, g1, g2, g3)
    wsem = (w0, w1, w2, w3)
    cid = lax.axis_index("c")
    sid = lax.axis_index("s")
    wid = sid * 2 + cid
    b = wid // 4
    per_w = _V // 4
    base = wid * per_w
    pltpu.sync_copy(cum_hbm.at[pl.ds(base, per_w)], cum_v)
    boff = b * _N
    nrounds = per_w // (4 * _GCH)

    def _round(r, carry):
        for t in range(4):
            k = r * 4 + t

            @pl.when(r > 0)
            def _():
                pltpu.make_async_copy(
                    rows_v.at[t],
                    g_hbm.at[pl.ds(base + (k - 4) * _GCH, _GCH)],
                    wsem[t],
                ).wait()

            def _lanes(j, c2, k=k, t=t):
                v = cum_v[pl.ds(k * _GCH + j * 16, 16)]
                idx_v[t, pl.ds(j * 16, 16)] = jnp.maximum(v - 1, 0) + boff
                return c2

            lax.fori_loop(0, _GCH // 16, _lanes, 0)
            pltpu.async_copy(s_hbm.at[idx_v.at[t]], rows_v.at[t], gsem[t])
        for t in range(4):
            k = r * 4 + t
            pltpu.make_async_copy(s_hbm.at[idx_v.at[t]], rows_v.at[t], gsem[t]).wait()
            pltpu.async_copy(
                rows_v.at[t], g_hbm.at[pl.ds(base + k * _GCH, _GCH)], wsem[t]
            )
        return carry

    lax.fori_loop(0, nrounds, _round, 0)
    for t in range(4):
        k = (nrounds - 1) * 4 + t
        pltpu.make_async_copy(
            rows_v.at[t], g_hbm.at[pl.ds(base + k * _GCH, _GCH)], wsem[t]
        ).wait()


def _scan_body(feat_ref, tri_ref, s_ref, carry_ref):
    j = pl.program_id(1)

    @pl.when(j == 0)
    def _():
        carry_ref[...] = jnp.zeros((1, _C), jnp.float32)

    x = feat_ref[0]
    tri = tri_ref[...]
    sc = lax.dot_general(
        tri, x,
        dimension_numbers=(((1,), (1,)), ((), ())),
        precision=lax.Precision.HIGHEST,
        preferred_element_type=jnp.float32,
    )
    sc = sc + carry_ref[...]
    s_ref[0] = sc
    carry_ref[...] = sc[_K - 1:_K, :]


def _fin_body(g_ref, gp_ref, cum_ref, cump_ref, eye_ref, out_ref):
    j = pl.program_id(1)
    x = g_ref[0]
    eye = eye_ref[...]
    xT = lax.dot_general(
        eye, x, dimension_numbers=(((1,), (1,)), ((), ())),
        precision=lax.Precision.HIGHEST,
        preferred_element_type=jnp.float32,
    )
    xpT = lax.dot_general(
        eye, gp_ref[0],
        dimension_numbers=(((1,), (1,)), ((), ())),
        precision=lax.Precision.HIGHEST,
        preferred_element_type=jnp.float32,
    )
    cum = cum_ref[0]
    cum_pl = jnp.where(j == 0, 0, cump_ref[0, 0, 0])
    cum_m1 = jnp.concatenate(
        [jnp.full((1, 1), cum_pl, jnp.int32), cum[:, :_KV - 1]], axis=1
    )
    hist = cum - cum_m1
    valid = hist > 0
    xTm1 = jnp.concatenate([xpT, xT[:, :_KV - 1]], axis=1)
    gm1 = jnp.where(cum_m1 >= 1, xTm1, 0.0)
    gcur = jnp.where(cum >= 1, xT, 0.0)
    mean = (gcur - gm1) / jnp.maximum(hist, 1).astype(jnp.float32)
    out_ref[0] = jnp.where(valid, mean, 0.0)


def kernel(pts, feat):
    pts_d = lax.stop_gradient(pts)
    pn = pts_d - jnp.min(pts_d, axis=2, keepdims=True)
    pn = pn / (jnp.max(jnp.linalg.norm(pn, axis=1)) + _EPS)
    vi3 = (pn * (_RES - 1)).astype(jnp.int32)
    vox = vi3[:, 0, :] + vi3[:, 1, :] * _RES + vi3[:, 2, :] * _RES * _RES
    vox1 = vox.reshape(_B * _N)

    cum1 = _sc_hist_cum(vox1)

    tri = jnp.tri(_K, dtype=jnp.float32)
    s = pl.pallas_call(
        _scan_body,
        grid=(_B, _N // _K),
        in_specs=[
            pl.BlockSpec((1, _C, _K), lambda b, j: (b, 0, j)),
            pl.BlockSpec((_K, _K), lambda b, j: (0, 0)),
        ],
        out_specs=pl.BlockSpec((1, _K, _C), lambda b, j: (b, j, 0)),
        out_shape=jax.ShapeDtypeStruct((_B, _N, _C), jnp.float32),
        scratch_shapes=[pltpu.VMEM((1, _C), jnp.float32)],
    )(feat, tri)

    g = _sc_gather(s.reshape(_B * _N, _C), cum1)

    nb = _V // _KV
    cumr = cum1.reshape(_B * nb, 1, _KV)
    g3 = g.reshape(_B, _V, _C)
    glast = g3[:, _KV - 1::_KV, :].reshape(_B * nb, 1, _C)
    cumlast = cum1.reshape(_B * nb, _KV)[:, _KV - 1:].reshape(_B * nb, 1, 1)
    eye = jnp.eye(_C, dtype=jnp.float32)
    out = pl.pallas_call(
        _fin_body,
        grid=(_B, _V // _KV),
        in_specs=[
            pl.BlockSpec((1, _KV, _C), lambda b, j: (b, j, 0)),
            pl.BlockSpec((1, 1, _C), lambda b, j: (b * nb + jnp.maximum(j - 1, 0), 0, 0)),
            pl.BlockSpec((1, 1, _KV), lambda b, j: (b * nb + j, 0, 0)),
            pl.BlockSpec((1, 1, 1), lambda b, j: (b * nb + jnp.maximum(j - 1, 0), 0, 0)),
            pl.BlockSpec((_C, _C), lambda b, j: (0, 0)),
        ],
        out_specs=pl.BlockSpec((1, _C, _KV), lambda b, j: (b, 0, j)),
        out_shape=jax.ShapeDtypeStruct((_B, _C, _V), jnp.float32),
    )(g3, glast, cumr, cumlast, eye)

    return out.reshape(_B, _C, _RES, _RES, _RES)


# speedup vs baseline: 1.0991x; 1.0991x over previous
import functools

import jax
import jax.numpy as jnp
from jax import lax
from jax.experimental import pallas as pl
from jax.experimental.pallas import tpu as pltpu
from jax.experimental.pallas import tpu_sc as plsc

_RES = 32
_EPS = 1e-08
_B = 8
_N = 32768
_C = 64
_V = _RES ** 3

_K = 512
_KBLK = 4096
_KV = 4096
_GCH = 128

_SC_MESH = plsc.VectorSubcoreMesh(core_axis_name="c", subcore_axis_name="s")


@functools.partial(
    pl.kernel,
    mesh=_SC_MESH,
    compiler_params=pltpu.CompilerParams(needs_layout_passes=False),
    out_type=jax.ShapeDtypeStruct((_B * _V,), jnp.int32),
    scratch_types=[
        pltpu.VMEM((_N,), jnp.int32),
        pltpu.VMEM((_V,), jnp.int32),
    ],
)
def _sc_hist_cum(vox_hbm, cum_hbm, buf_v, hist_v):
    cid = lax.axis_index("c")
    sid = lax.axis_index("s")
    wid = sid * 2 + cid

    @pl.when(wid < _B)
    def _():
        b = wid
        pltpu.sync_copy(vox_hbm.at[pl.ds(b * _N, _N)], buf_v)

        zeros16 = jnp.zeros((16,), jnp.int32)

        def _zero(i, carry):
            hist_v[pl.ds(i * 16, 16)] = zeros16
            return carry

        lax.fori_loop(0, _V // 16, _zero, 0)

        ones16 = jnp.ones((16,), jnp.int32)
        lane = lax.broadcasted_iota(jnp.int32, (16,), 0)

        def _hist(i, carry):
            idx = buf_v[pl.ds(i * 16, 16)]
            for j in range(16):
                plsc.addupdate_scatter(hist_v, [idx], ones16, mask=lane == j)
            return carry

        lax.fori_loop(0, _N // 16, _hist, 0)

        def _cum(i, carry):
            v = hist_v[pl.ds(i * 16, 16)]
            cs = jnp.cumsum(v) + carry
            buf_v[pl.ds(i * 16, 16)] = cs
            return carry + jnp.sum(v)

        lax.fori_loop(0, _V // 16, _cum, jnp.int32(0))
        pltpu.sync_copy(buf_v, cum_hbm.at[pl.ds(b * _V, _V)])


@functools.partial(
    pl.kernel,
    mesh=_SC_MESH,
    compiler_params=pltpu.CompilerParams(
        needs_layout_passes=False, use_tc_tiling_on_sc=False
    ),
    out_type=jax.ShapeDtypeStruct((_B * _V, _C), jnp.float32),
    scratch_types=[
        pltpu.VMEM((_V // 4,), jnp.int32),
        pltpu.VMEM((4, _GCH), jnp.int32),
        pltpu.VMEM((4, _GCH, _C), jnp.float32),
        pltpu.SemaphoreType.DMA,
        pltpu.SemaphoreType.DMA,
        pltpu.SemaphoreType.DMA,
        pltpu.SemaphoreType.DMA,
        pltpu.SemaphoreType.DMA,
        pltpu.SemaphoreType.DMA,
        pltpu.SemaphoreType.DMA,
        pltpu.SemaphoreType.DMA,
    ],
)
def _sc_gather(s_hbm, cum_hbm, g_hbm, cum_v, idx_v, rows_v,
               g0, g1, g2, g3, w0, w1, w2, w3):
    gsem = (g0, g1, g2, g3)
    wsem = (w0, w1, w2, w3)
    cid = lax.axis_index("c")
    sid = lax.axis_index("s")
    wid = sid * 2 + cid
    b = wid // 4
    per_w = _V // 4
    base = wid * per_w
    pltpu.sync_copy(cum_hbm.at[pl.ds(base, per_w)], cum_v)
    boff = b * _N
    nrounds = per_w // (4 * _GCH)

    def _round(r, carry):
        for t in range(4):
            k = r * 4 + t

            @pl.when(r > 0)
            def _():
                pltpu.make_async_copy(
                    rows_v.at[t],
                    g_hbm.at[pl.ds(base + (k - 4) * _GCH, _GCH)],
                    wsem[t],
                ).wait()

            def _lanes(j, c2, k=k, t=t):
                v = cum_v[pl.ds(k * _GCH + j * 16, 16)]
                idx_v[t, pl.ds(j * 16, 16)] = jnp.maximum(v - 1, 0) + boff
                return c2

            lax.fori_loop(0, _GCH // 16, _lanes, 0)
            pltpu.async_copy(s_hbm.at[idx_v.at[t]], rows_v.at[t], gsem[t])
        for t in range(4):
            k = r * 4 + t
            pltpu.make_async_copy(s_hbm.at[idx_v.at[t]], rows_v.at[t], gsem[t]).wait()
            pltpu.async_copy(
                rows_v.at[t], g_hbm.at[pl.ds(base + k * _GCH, _GCH)], wsem[t]
            )
        return carry

    lax.fori_loop(0, nrounds, _round, 0)
    for t in range(4):
        k = (nrounds - 1) * 4 + t
        pltpu.make_async_copy(
            rows_v.at[t], g_hbm.at[pl.ds(base + k * _GCH, _GCH)], wsem[t]
        ).wait()


def _scan_body(feat_ref, tri_ref, s_ref, carry_ref):
    j = pl.program_id(1)

    @pl.when(j == 0)
    def _():
        carry_ref[...] = jnp.zeros((1, _C), jnp.float32)

    tri = tri_ref[...]
    c = carry_ref[...]
    for t in range(_KBLK // _K):
        x = feat_ref[0, :, t * _K:(t + 1) * _K]
        sc = lax.dot_general(
            tri, x,
            dimension_numbers=(((1,), (1,)), ((), ())),
            precision=lax.Precision.HIGHEST,
            preferred_element_type=jnp.float32,
        )
        sc = sc + c
        s_ref[0, t * _K:(t + 1) * _K, :] = sc
        c = sc[_K - 1:_K, :]
    carry_ref[...] = c


def _fin_body(g_ref, gp_ref, cum_ref, cump_ref, eye_ref, out_ref):
    j = pl.program_id(1)
    x = g_ref[0]
    eye = eye_ref[...]
    xT = lax.dot_general(
        eye, x, dimension_numbers=(((1,), (1,)), ((), ())),
        precision=lax.Precision.HIGHEST,
        preferred_element_type=jnp.float32,
    )
    xpT = lax.dot_general(
        eye, gp_ref[0],
        dimension_numbers=(((1,), (1,)), ((), ())),
        precision=lax.Precision.HIGHEST,
        preferred_element_type=jnp.float32,
    )
    cum = cum_ref[0]
    cum_pl = jnp.where(j == 0, 0, cump_ref[0, 0, 0])
    cum_m1 = jnp.concatenate(
        [jnp.full((1, 1), cum_pl, jnp.int32), cum[:, :_KV - 1]], axis=1
    )
    hist = cum - cum_m1
    valid = hist > 0
    xTm1 = jnp.concatenate([xpT, xT[:, :_KV - 1]], axis=1)
    gm1 = jnp.where(cum_m1 >= 1, xTm1, 0.0)
    gcur = jnp.where(cum >= 1, xT, 0.0)
    mean = (gcur - gm1) / jnp.maximum(hist, 1).astype(jnp.float32)
    out_ref[0] = jnp.where(valid, mean, 0.0)


def kernel(pts, feat):
    pts_d = lax.stop_gradient(pts)
    pn = pts_d - jnp.min(pts_d, axis=2, keepdims=True)
    pn = pn / (jnp.max(jnp.linalg.norm(pn, axis=1)) + _EPS)
    vi3 = (pn * (_RES - 1)).astype(jnp.int32)
    vox = vi3[:, 0, :] + vi3[:, 1, :] * _RES + vi3[:, 2, :] * _RES * _RES
    vox1 = vox.reshape(_B * _N)

    cum1 = _sc_hist_cum(vox1)

    tri = jnp.tri(_K, dtype=jnp.float32)
    s = pl.pallas_call(
        _scan_body,
        grid=(_B, _N // _KBLK),
        in_specs=[
            pl.BlockSpec((1, _C, _KBLK), lambda b, j: (b, 0, j)),
            pl.BlockSpec((_K, _K), lambda b, j: (0, 0)),
        ],
        out_specs=pl.BlockSpec((1, _KBLK, _C), lambda b, j: (b, j, 0)),
        out_shape=jax.ShapeDtypeStruct((_B, _N, _C), jnp.float32),
        scratch_shapes=[pltpu.VMEM((1, _C), jnp.float32)],
    )(feat, tri)

    g = _sc_gather(s.reshape(_B * _N, _C), cum1)

    nb = _V // _KV
    cumr = cum1.reshape(_B * nb, 1, _KV)
    g3 = g.reshape(_B, _V, _C)
    glast = g3[:, _KV - 1::_KV, :].reshape(_B * nb, 1, _C)
    cumlast = cum1.reshape(_B * nb, _KV)[:, _KV - 1:].reshape(_B * nb, 1, 1)
    eye = jnp.eye(_C, dtype=jnp.float32)
    out = pl.pallas_call(
        _fin_body,
        grid=(_B, _V // _KV),
        in_specs=[
            pl.BlockSpec((1, _KV, _C), lambda b, j: (b, j, 0)),
            pl.BlockSpec((1, 1, _C), lambda b, j: (b * nb + jnp.maximum(j - 1, 0), 0, 0)),
            pl.BlockSpec((1, 1, _KV), lambda b, j: (b * nb + j, 0, 0)),
            pl.BlockSpec((1, 1, 1), lambda b, j: (b * nb + jnp.maximum(j - 1, 0), 0, 0)),
            pl.BlockSpec((_C, _C), lambda b, j: (0, 0)),
        ],
        out_specs=pl.BlockSpec((1, _C, _KV), lambda b, j: (b, 0, j)),
        out_shape=jax.ShapeDtypeStruct((_B, _C, _V), jnp.float32),
    )(g3, glast, cumr, cumlast, eye)

    return out.reshape(_B, _C, _RES, _RES, _RES)


# speedup vs baseline: 1.1139x; 1.0135x over previous
import functools

import jax
import jax.numpy as jnp
from jax import lax
from jax.experimental import pallas as pl
from jax.experimental.pallas import tpu as pltpu
from jax.experimental.pallas import tpu_sc as plsc

_RES = 32
_EPS = 1e-08
_B = 8
_N = 32768
_C = 64
_V = _RES ** 3

_K = 512
_KBLK = 8192
_KV = 8192
_GCH = 128

_SC_MESH = plsc.VectorSubcoreMesh(core_axis_name="c", subcore_axis_name="s")


@functools.partial(
    pl.kernel,
    mesh=_SC_MESH,
    compiler_params=pltpu.CompilerParams(needs_layout_passes=False),
    out_type=jax.ShapeDtypeStruct((_B * _V,), jnp.int32),
    scratch_types=[
        pltpu.VMEM((_N,), jnp.int32),
        pltpu.VMEM((_V,), jnp.int32),
    ],
)
def _sc_hist_cum(vox_hbm, cum_hbm, buf_v, hist_v):
    cid = lax.axis_index("c")
    sid = lax.axis_index("s")
    wid = sid * 2 + cid

    @pl.when(wid < _B)
    def _():
        b = wid
        pltpu.sync_copy(vox_hbm.at[pl.ds(b * _N, _N)], buf_v)

        zeros16 = jnp.zeros((16,), jnp.int32)

        def _zero(i, carry):
            hist_v[pl.ds(i * 16, 16)] = zeros16
            return carry

        lax.fori_loop(0, _V // 16, _zero, 0)

        ones16 = jnp.ones((16,), jnp.int32)
        lane = lax.broadcasted_iota(jnp.int32, (16,), 0)

        def _hist(i, carry):
            idx = buf_v[pl.ds(i * 16, 16)]
            for j in range(16):
                plsc.addupdate_scatter(hist_v, [idx], ones16, mask=lane == j)
            return carry

        lax.fori_loop(0, _N // 16, _hist, 0)

        def _cum(i, carry):
            v = hist_v[pl.ds(i * 16, 16)]
            cs = jnp.cumsum(v) + carry
            buf_v[pl.ds(i * 16, 16)] = cs
            return carry + jnp.sum(v)

        lax.fori_loop(0, _V // 16, _cum, jnp.int32(0))
        pltpu.sync_copy(buf_v, cum_hbm.at[pl.ds(b * _V, _V)])


@functools.partial(
    pl.kernel,
    mesh=_SC_MESH,
    compiler_params=pltpu.CompilerParams(
        needs_layout_passes=False, use_tc_tiling_on_sc=False
    ),
    out_type=jax.ShapeDtypeStruct((_B * _V, _C), jnp.float32),
    scratch_types=[
        pltpu.VMEM((_V // 4,), jnp.int32),
        pltpu.VMEM((4, _GCH), jnp.int32),
        pltpu.VMEM((4, _GCH, _C), jnp.float32),
        pltpu.SemaphoreType.DMA,
        pltpu.SemaphoreType.DMA,
        pltpu.SemaphoreType.DMA,
        pltpu.SemaphoreType.DMA,
        pltpu.SemaphoreType.DMA,
        pltpu.SemaphoreType.DMA,
        pltpu.SemaphoreType.DMA,
        pltpu.SemaphoreType.DMA,
    ],
)
def _sc_gather(s_hbm, cum_hbm, g_hbm, cum_v, idx_v, rows_v,
               g0, g1, g2, g3, w0, w1, w2, w3):
    gsem = (g0, g1, g2, g3)
    wsem = (w0, w1, w2, w3)
    cid = lax.axis_index("c")
    sid = lax.axis_index("s")
    wid = sid * 2 + cid
    b = wid // 4
    per_w = _V // 4
    base = wid * per_w
    pltpu.sync_copy(cum_hbm.at[pl.ds(base, per_w)], cum_v)
    boff = b * _N
    nrounds = per_w // (4 * _GCH)

    def _round(r, carry):
        for t in range(4):
            k = r * 4 + t

            @pl.when(r > 0)
            def _():
                pltpu.make_async_copy(
                    rows_v.at[t],
                    g_hbm.at[pl.ds(base + (k - 4) * _GCH, _GCH)],
                    wsem[t],
                ).wait()

            def _lanes(j, c2, k=k, t=t):
                v = cum_v[pl.ds(k * _GCH + j * 16, 16)]
                idx_v[t, pl.ds(j * 16, 16)] = jnp.maximum(v - 1, 0) + boff
                return c2

            lax.fori_loop(0, _GCH // 16, _lanes, 0)
            pltpu.async_copy(s_hbm.at[idx_v.at[t]], rows_v.at[t], gsem[t])
        for t in range(4):
            k = r * 4 + t
            pltpu.make_async_copy(s_hbm.at[idx_v.at[t]], rows_v.at[t], gsem[t]).wait()
            pltpu.async_copy(
                rows_v.at[t], g_hbm.at[pl.ds(base + k * _GCH, _GCH)], wsem[t]
            )
        return carry

    lax.fori_loop(0, nrounds, _round, 0)
    for t in range(4):
        k = (nrounds - 1) * 4 + t
        pltpu.make_async_copy(
            rows_v.at[t], g_hbm.at[pl.ds(base + k * _GCH, _GCH)], wsem[t]
        ).wait()


def _scan_body(feat_ref, tri_ref, s_ref, carry_ref):
    j = pl.program_id(1)

    @pl.when(j == 0)
    def _():
        carry_ref[...] = jnp.zeros((1, _C), jnp.float32)

    tri = tri_ref[...]
    c = carry_ref[...]
    for t in range(_KBLK // _K):
        x = feat_ref[0, :, t * _K:(t + 1) * _K]
        sc = lax.dot_general(
            tri, x,
            dimension_numbers=(((1,), (1,)), ((), ())),
            precision=lax.Precision.HIGHEST,
            preferred_element_type=jnp.float32,
        )
        sc = sc + c
        s_ref[0, t * _K:(t + 1) * _K, :] = sc
        c = sc[_K - 1:_K, :]
    carry_ref[...] = c


def _fin_body(g_ref, gp_ref, cum_ref, cump_ref, eye_ref, out_ref):
    j = pl.program_id(1)
    x = g_ref[0]
    eye = eye_ref[...]
    xT = lax.dot_general(
        eye, x, dimension_numbers=(((1,), (1,)), ((), ())),
        precision=lax.Precision.HIGHEST,
        preferred_element_type=jnp.float32,
    )
    xpT = lax.dot_general(
        eye, gp_ref[0],
        dimension_numbers=(((1,), (1,)), ((), ())),
        precision=lax.Precision.HIGHEST,
        preferred_element_type=jnp.float32,
    )
    cum = cum_ref[0]
    cum_pl = jnp.where(j == 0, 0, cump_ref[0, 0, 0])
    cum_m1 = jnp.concatenate(
        [jnp.full((1, 1), cum_pl, jnp.int32), cum[:, :_KV - 1]], axis=1
    )
    hist = cum - cum_m1
    valid = hist > 0
    xTm1 = jnp.concatenate([xpT, xT[:, :_KV - 1]], axis=1)
    gm1 = jnp.where(cum_m1 >= 1, xTm1, 0.0)
    gcur = jnp.where(cum >= 1, xT, 0.0)
    mean = (gcur - gm1) / jnp.maximum(hist, 1).astype(jnp.float32)
    out_ref[0] = jnp.where(valid, mean, 0.0)


def kernel(pts, feat):
    pts_d = lax.stop_gradient(pts)
    pn = pts_d - jnp.min(pts_d, axis=2, keepdims=True)
    pn = pn / (jnp.max(jnp.linalg.norm(pn, axis=1)) + _EPS)
    vi3 = (pn * (_RES - 1)).astype(jnp.int32)
    vox = vi3[:, 0, :] + vi3[:, 1, :] * _RES + vi3[:, 2, :] * _RES * _RES
    vox1 = vox.reshape(_B * _N)

    cum1 = _sc_hist_cum(vox1)

    tri = jnp.tri(_K, dtype=jnp.float32)
    s = pl.pallas_call(
        _scan_body,
        grid=(_B, _N // _KBLK),
        in_specs=[
            pl.BlockSpec((1, _C, _KBLK), lambda b, j: (b, 0, j)),
            pl.BlockSpec((_K, _K), lambda b, j: (0, 0)),
        ],
        out_specs=pl.BlockSpec((1, _KBLK, _C), lambda b, j: (b, j, 0)),
        out_shape=jax.ShapeDtypeStruct((_B, _N, _C), jnp.float32),
        scratch_shapes=[pltpu.VMEM((1, _C), jnp.float32)],
    )(feat, tri)

    g = _sc_gather(s.reshape(_B * _N, _C), cum1)

    nb = _V // _KV
    cumr = cum1.reshape(_B * nb, 1, _KV)
    g3 = g.reshape(_B, _V, _C)
    glast = g3[:, _KV - 1::_KV, :].reshape(_B * nb, 1, _C)
    cumlast = cum1.reshape(_B * nb, _KV)[:, _KV - 1:].reshape(_B * nb, 1, 1)
    eye = jnp.eye(_C, dtype=jnp.float32)
    out = pl.pallas_call(
        _fin_body,
        grid=(_B, _V // _KV),
        in_specs=[
            pl.BlockSpec((1, _KV, _C), lambda b, j: (b, j, 0)),
            pl.BlockSpec((1, 1, _C), lambda b, j: (b * nb + jnp.maximum(j - 1, 0), 0, 0)),
            pl.BlockSpec((1, 1, _KV), lambda b, j: (b * nb + j, 0, 0)),
            pl.BlockSpec((1, 1, 1), lambda b, j: (b * nb + jnp.maximum(j - 1, 0), 0, 0)),
            pl.BlockSpec((_C, _C), lambda b, j: (0, 0)),
        ],
        out_specs=pl.BlockSpec((1, _C, _KV), lambda b, j: (b, 0, j)),
        out_shape=jax.ShapeDtypeStruct((_B, _C, _V), jnp.float32),
    )(g3, glast, cumr, cumlast, eye)

    return out.reshape(_B, _C, _RES, _RES, _RES)
